# baseline (speedup 1.0000x reference)
import functools

import jax
import jax.numpy as jnp
from jax.experimental import pallas as pl
from jax.experimental.pallas import tpu as pltpu
from jax.experimental.pallas import tpu_sc as plsc

TB = 512
FB = 512
PB = 256
GW = 128


def _sc_gather_rows(src, idx, chunk):
    n = idx.shape[0]
    d = src.shape[1]
    nd = d // chunk
    src2 = src.reshape(src.shape[0] * nd, chunk)
    idx2 = (idx[:, None] * nd + jnp.arange(nd, dtype=jnp.int32)[None, :])
    idx2 = idx2.reshape(1, n * nd)
    mesh = plsc.VectorSubcoreMesh(core_axis_name="c", subcore_axis_name="s")

    @functools.partial(
        pl.kernel,
        out_type=jax.ShapeDtypeStruct((n * nd, chunk), src.dtype),
        mesh=mesh,
    )
    def gather_kernel(src_hbm, idx_hbm, out_hbm):
        def body(idx_vmem, out_vmem):
            pltpu.sync_copy(src_hbm.at[idx_vmem.at[0]], out_vmem)

        pltpu.emit_pipeline(
            body,
            grid=(n * nd // GW,),
            in_specs=[pl.BlockSpec((1, GW), lambda i: (0, i))],
            out_specs=[pl.BlockSpec((GW, chunk), lambda i: (i, 0))],
            core_axis_name=("c", "s"),
            dimension_semantics=(pltpu.PARALLEL,),
        )(idx_hbm, out_hbm)

    return gather_kernel(src2, idx2).reshape(n, d)


def _min_idx(i, m):
    return jnp.minimum(i, m)


def _prep_body(f, fb, axis, a_ref, b_ref, o_ref):
    c = pl.program_id(0)
    limit = f - c * fb
    pos = jax.lax.broadcasted_iota(jnp.int32, o_ref.shape[2:], axis)
    keep = pos < limit
    o_ref[0, 0] = jnp.where(keep, a_ref[...], 0.0).astype(jnp.bfloat16)
    o_ref[1, 0] = jnp.where(keep, b_ref[...], 0.0).astype(jnp.bfloat16)


def _stack_cast_pad(a, b, nf, axis):
    f = a.shape[axis]
    d = a.shape[1 - axis]

    if axis == 1:
        in_spec = pl.BlockSpec((d, FB), lambda c: (0, c))
        out_spec = pl.BlockSpec((2, 1, d, FB), lambda c: (0, c, 0, 0))
        out_shape = jax.ShapeDtypeStruct((2, nf, d, FB), jnp.bfloat16)
    else:
        in_spec = pl.BlockSpec((FB, d), lambda c: (c, 0))
        out_spec = pl.BlockSpec((2, 1, FB, d), lambda c: (0, c, 0, 0))
        out_shape = jax.ShapeDtypeStruct((2, nf, FB, d), jnp.bfloat16)

    return pl.pallas_call(
        functools.partial(_prep_body, f, FB, axis),
        grid=(nf,),
        in_specs=[in_spec, in_spec],
        out_specs=out_spec,
        out_shape=out_shape,
        compiler_params=pltpu.CompilerParams(
            dimension_semantics=("arbitrary",),
        ),
    )(a, b)


def _mlp_body(nf, eid_ref, x_ref, gw_ref, uw_ref, dw_ref, y_ref, h_ref):
    s = pl.program_id(0)
    par = jax.lax.rem(s, 2)

    contrib = jnp.dot(
        h_ref[1 - par], dw_ref[0, 0], preferred_element_type=jnp.float32
    )
    first = jax.lax.rem(s - 1, nf) == 0
    y_ref[...] = jnp.where(first, contrib, y_ref[...] + contrib)

    x = x_ref[...]
    g = jnp.dot(x, gw_ref[0, 0], preferred_element_type=jnp.float32)
    u = jnp.dot(x, uw_ref[0, 0], preferred_element_type=jnp.float32)
    h_ref[par] = (jax.nn.silu(g) * u).astype(jnp.bfloat16)


def kernel(hidden_states, token_type_ids, vg_w, vu_w, vd_w, lg_w, lu_w, ld_w):
    B, L, D = hidden_states.shape
    F = vg_w.shape[1]
    N = B * L
    NP = N + TB
    NB = NP // TB
    FP = ((F + FB - 1) // FB) * FB
    NF = FP // FB

    tt = token_type_ids
    inner = (tt[:, :-1] == 1) & (tt[:, 1:] == 1)
    vmask = jnp.concatenate(
        [inner, jnp.zeros((B, 1), dtype=jnp.bool_)], axis=1
    ).reshape(N)
    mvi = vmask.astype(jnp.int32)
    vc = jnp.cumsum(mvi)
    nv = vc[-1]
    nv_pad = ((nv + TB - 1) // TB) * TB
    lc = jnp.cumsum(1 - mvi)
    dest = jnp.where(vmask, vc - 1, nv_pad + lc - 1).astype(jnp.int32)
    perm = jnp.zeros(NP, jnp.int32).at[dest].set(jnp.arange(N, dtype=jnp.int32))
    eids = (jnp.arange(NB, dtype=jnp.int32) * TB >= nv_pad).astype(jnp.int32)

    gw_s = _stack_cast_pad(vg_w, lg_w, NF, axis=1)
    uw_s = _stack_cast_pad(vu_w, lu_w, NF, axis=1)
    dw_s = _stack_cast_pad(vd_w, ld_w, NF, axis=0)

    x = hidden_states.reshape(N, D)
    x_sorted = _sc_gather_rows(x, perm, 256).astype(jnp.bfloat16)

    S = NB * NF + 1

    def _cur(s):
        return _min_idx(s, NB * NF - 1)

    def _prev(s):
        return jnp.maximum(s - 1, 0)

    grid_spec = pltpu.PrefetchScalarGridSpec(
        num_scalar_prefetch=1,
        grid=(S,),
        in_specs=[
            pl.BlockSpec((TB, D), lambda s, eid: (_cur(s) // NF, 0)),
            pl.BlockSpec(
                (1, 1, D, FB),
                lambda s, eid: (eid[_cur(s) // NF], _cur(s) % NF, 0, 0),
            ),
            pl.BlockSpec(
                (1, 1, D, FB),
                lambda s, eid: (eid[_cur(s) // NF], _cur(s) % NF, 0, 0),
            ),
            pl.BlockSpec(
                (1, 1, FB, D),
                lambda s, eid: (eid[_prev(s) // NF], _prev(s) % NF, 0, 0),
            ),
        ],
        out_specs=pl.BlockSpec((TB, D), lambda s, eid: (_prev(s) // NF, 0)),
        scratch_shapes=[pltpu.VMEM((2, TB, FB), jnp.bfloat16)],
    )
    y_sorted = pl.pallas_call(
        functools.partial(_mlp_body, NF),
        grid_spec=grid_spec,
        out_shape=jax.ShapeDtypeStruct((NP, D), jnp.float32),
        compiler_params=pltpu.CompilerParams(
            dimension_semantics=("arbitrary",),
        ),
    )(eids, x_sorted, gw_s, uw_s, dw_s)

    out = _sc_gather_rows(y_sorted, dest, 256)
    return out.reshape(B, L, D)
